# baseline (speedup 1.0000x reference)
import functools

import jax
import jax.numpy as jnp
from jax import lax
from jax.experimental import pallas as pl
from jax.experimental.pallas import tpu as pltpu
from jax.experimental.pallas import tpu_sc as plsc

TILE = 16
DIM_OUT = 128
W0 = 30.0
VOCAB = TILE * TILE

BATCH = 4
SEQ = 147456
B = BATCH * SEQ
NROW = B // 128

NW = 32
BPW = B // NW
CH = 128
NCH = BPW // CH


def _m_body(t_ref, w_ref, m_ref):
    m_ref[...] = W0 * lax.dot_general(
        t_ref[...], w_ref[...],
        dimension_numbers=(((1,), (1,)), ((), ())),
        preferred_element_type=jnp.float32,
    )


def _fused_table(table, W):
    return pl.pallas_call(
        _m_body,
        out_shape=jax.ShapeDtypeStruct((VOCAB, DIM_OUT), jnp.float32),
    )(table, W)


def _idx_body(x_ref, idx_ref):
    v = x_ref[...].astype(jnp.float32)
    lane = lax.broadcasted_iota(jnp.int32, (1, 2 * 128), 1)
    pat = jnp.where(lane % 2 == 0, 1.0, float(TILE))
    w = v * pat
    jj = lax.broadcasted_iota(jnp.int32, (2 * 128, 128), 0)
    kk = lax.broadcasted_iota(jnp.int32, (2 * 128, 128), 1)
    sel = (jj // 2 == kk).astype(jnp.float32)
    idx_f = lax.dot_general(
        w, sel,
        dimension_numbers=(((1,), (0,)), ((), ())),
        preferred_element_type=jnp.float32,
    )
    idx_ref[...] = idx_f.astype(jnp.int32)


def _indices(xr):
    bs = 512
    return pl.pallas_call(
        _idx_body,
        grid=(NROW // bs,),
        in_specs=[pl.BlockSpec((bs, 2 * 128), lambda i: (i, 0))],
        out_specs=pl.BlockSpec((bs, 128), lambda i: (i, 0)),
        out_shape=jax.ShapeDtypeStruct((NROW, 128), jnp.int32),
    )(xr)


NBUF = 3
CHW = CH * DIM_OUT


@functools.cache
def _sc_gather_kernel():
    @functools.partial(
        pl.kernel,
        mesh=plsc.VectorSubcoreMesh(
            core_axis_name="c", subcore_axis_name="s", num_cores=2
        ),
        out_type=jax.ShapeDtypeStruct((B, DIM_OUT), jnp.float32),
        compiler_params=pltpu.CompilerParams(needs_layout_passes=False),
        scratch_types=[
            pltpu.VMEM((VOCAB, DIM_OUT), jnp.float32),
            pltpu.VMEM((NCH, CH), jnp.int32),
            *[pltpu.VMEM((CH, DIM_OUT), jnp.float32) for _ in range(NBUF)],
            pltpu.SemaphoreType.DMA,
            *[pltpu.SemaphoreType.DMA for _ in range(NBUF)],
        ],
    )
    def _sc_gather(m_hbm, idx_hbm, out_hbm, m_v, idx_v, *rest):
        bufs = rest[:NBUF]
        lsem = rest[NBUF]
        ssems = rest[NBUF + 1:]
        wid = lax.axis_index("s") * 2 + lax.axis_index("c")
        base = wid * BPW
        pltpu.sync_copy(idx_hbm.at[pl.ds(wid * NCH, NCH)], idx_v)
        pltpu.async_copy(m_hbm, m_v, lsem).wait()

        iota = lax.iota(jnp.int32, 16)
        cols = [jnp.full((16,), c, jnp.int32) for c in range(DIM_OUT)]

        def compute_chunk(jj, buf):
            def gbody(g, carry):
                rows = idx_v[jj, pl.ds(pl.multiple_of(g * 16, 16), 16)]
                wrows = g * 16 + iota
                for c in range(DIM_OUT):
                    v = plsc.load_gather(m_v, [rows, cols[c]])
                    plsc.store_scatter(buf, [wrows, cols[c]], v)
                return carry
            lax.fori_loop(0, CH // 16, gbody, 0)

        def body(r, carry):
            for b in range(NBUF):
                jj = r * NBUF + b

                @pl.when(r > 0)
                def _():
                    pltpu.make_async_copy(
                        bufs[b],
                        out_hbm.at[pl.ds(base + (jj - NBUF) * CH, CH)],
                        ssems[b]).wait()

                compute_chunk(jj, bufs[b])
                pltpu.async_copy(
                    bufs[b], out_hbm.at[pl.ds(base + jj * CH, CH)], ssems[b])
            return carry

        lax.fori_loop(0, NCH // NBUF, body, 0)

        for b in range(NBUF):
            pltpu.make_async_copy(
                bufs[b],
                out_hbm.at[pl.ds(base + (NCH - NBUF + b) * CH, CH)],
                ssems[b]).wait()

    return _sc_gather


def kernel(x, table, W):
    M = _fused_table(table, W)
    xr = x.reshape(NROW, 2 * 128)
    idx2 = _indices(xr)
    out = _sc_gather_kernel()(M, idx2)
    return out.reshape(BATCH, SEQ, DIM_OUT)


# speedup vs baseline: 3.3807x; 3.3807x over previous
import functools

import jax
import jax.numpy as jnp
from jax import lax
from jax.experimental import pallas as pl
from jax.experimental.pallas import tpu as pltpu
from jax.experimental.pallas import tpu_sc as plsc

TILE = 16
DIM_OUT = 128
W0 = 30.0
VOCAB = TILE * TILE

BATCH = 4
SEQ = 147456
B = BATCH * SEQ
NROW = B // 128

NW = 32
BPW = B // NW
CH = 128
NCH = BPW // CH


def _m_body(t_ref, w_ref, m_ref):
    m_ref[...] = W0 * lax.dot_general(
        t_ref[...], w_ref[...],
        dimension_numbers=(((1,), (1,)), ((), ())),
        preferred_element_type=jnp.float32,
    )


def _fused_table(table, W):
    return pl.pallas_call(
        _m_body,
        out_shape=jax.ShapeDtypeStruct((VOCAB, DIM_OUT), jnp.float32),
    )(table, W)


def _idx_body(x_ref, idx_ref):
    v = x_ref[...].astype(jnp.float32)
    lane = lax.broadcasted_iota(jnp.int32, (1, 2 * 128), 1)
    pat = jnp.where(lane % 2 == 0, 1.0, float(TILE))
    w = v * pat
    jj = lax.broadcasted_iota(jnp.int32, (2 * 128, 128), 0)
    kk = lax.broadcasted_iota(jnp.int32, (2 * 128, 128), 1)
    sel = (jj // 2 == kk).astype(jnp.float32)
    idx_f = lax.dot_general(
        w, sel,
        dimension_numbers=(((1,), (0,)), ((), ())),
        preferred_element_type=jnp.float32,
    )
    idx_ref[...] = idx_f.astype(jnp.int32)


def _indices(xr):
    bs = 512
    return pl.pallas_call(
        _idx_body,
        grid=(NROW // bs,),
        in_specs=[pl.BlockSpec((bs, 2 * 128), lambda i: (i, 0))],
        out_specs=pl.BlockSpec((bs, 128), lambda i: (i, 0)),
        out_shape=jax.ShapeDtypeStruct((NROW, 128), jnp.int32),
    )(xr)


NBUF = 3
CHW = CH * DIM_OUT


@functools.cache
def _sc_gather_kernel():
    @functools.partial(
        pl.kernel,
        mesh=plsc.VectorSubcoreMesh(
            core_axis_name="c", subcore_axis_name="s", num_cores=2
        ),
        out_type=jax.ShapeDtypeStruct((B * DIM_OUT,), jnp.float32),
        compiler_params=pltpu.CompilerParams(needs_layout_passes=False),
        scratch_types=[
            pltpu.VMEM((VOCAB * DIM_OUT,), jnp.float32),
            pltpu.VMEM((NCH, CH), jnp.int32),
            *[pltpu.VMEM((CHW,), jnp.float32) for _ in range(NBUF)],
            pltpu.SemaphoreType.DMA,
            *[pltpu.SemaphoreType.DMA for _ in range(NBUF)],
        ],
    )
    def _sc_gather(m_hbm, idx_hbm, out_hbm, m_v, idx_v, *rest):
        bufs = rest[:NBUF]
        lsem = rest[NBUF]
        ssems = rest[NBUF + 1:]
        wid = lax.axis_index("s") * 2 + lax.axis_index("c")
        base = wid * BPW * DIM_OUT
        pltpu.sync_copy(idx_hbm.at[pl.ds(wid * NCH, NCH)], idx_v)
        pltpu.async_copy(m_hbm, m_v, lsem).wait()

        iota = lax.iota(jnp.int32, 16)
        diag = [(iota + k) & 15 for k in range(16)]

        def body(r, carry):
            for b in range(NBUF):
                jj = r * NBUF + b

                @pl.when(r > 0)
                def _():
                    pltpu.make_async_copy(
                        bufs[b],
                        out_hbm.at[pl.ds(base + (jj - NBUF) * CHW, CHW)],
                        ssems[b]).wait()

                buf = bufs[b]

                @plsc.parallel_loop(0, CH // 16, unroll=1)
                def _(g):
                    rows = idx_v[jj, pl.ds(pl.multiple_of(g * 16, 16), 16)]
                    raddr = rows * DIM_OUT
                    waddr = (g * 16 + iota) * DIM_OUT
                    for t in range(DIM_OUT // 16):
                        for k in range(16):
                            c = diag[k] + t * 16
                            v = plsc.load_gather(m_v, [raddr + c])
                            plsc.store_scatter(buf, [waddr + c], v)

                pltpu.async_copy(
                    bufs[b], out_hbm.at[pl.ds(base + jj * CHW, CHW)],
                    ssems[b])
            return carry

        lax.fori_loop(0, NCH // NBUF, body, 0)

        for b in range(NBUF):
            pltpu.make_async_copy(
                bufs[b],
                out_hbm.at[pl.ds(base + (NCH - NBUF + b) * CHW, CHW)],
                ssems[b]).wait()

    return _sc_gather


def kernel(x, table, W):
    M = _fused_table(table, W)
    xr = x.reshape(NROW, 2 * 128)
    idx2 = _indices(xr)
    out = _sc_gather_kernel()(M.reshape(-1), idx2)
    return out.reshape(BATCH, SEQ, DIM_OUT)


# speedup vs baseline: 4.0590x; 1.2006x over previous
import functools

import jax
import jax.numpy as jnp
from jax import lax
from jax.experimental import pallas as pl
from jax.experimental.pallas import tpu as pltpu
from jax.experimental.pallas import tpu_sc as plsc

TILE = 16
DIM_OUT = 128
W0 = 30.0
VOCAB = TILE * TILE

BATCH = 4
SEQ = 147456
B = BATCH * SEQ
NROW = B // 128

NW = 32
RT = 393216
BSC = B - RT
BPW = BSC // NW
CH = 128
NCH = BPW // CH
KC = 128


def _m_body(t_ref, w_ref, m_ref):
    m_ref[...] = W0 * lax.dot_general(
        t_ref[...], w_ref[...],
        dimension_numbers=(((1,), (1,)), ((), ())),
        preferred_element_type=jnp.float32,
    )


def _fused_table(table, W):
    return pl.pallas_call(
        _m_body,
        out_shape=jax.ShapeDtypeStruct((VOCAB, DIM_OUT), jnp.float32),
    )(table, W)


def _idx_body(x_ref, idx_ref):
    v = x_ref[...].astype(jnp.float32)
    lane = lax.broadcasted_iota(jnp.int32, (1, 2 * 128), 1)
    pat = jnp.where(lane % 2 == 0, 1.0, float(TILE))
    w = v * pat
    jj = lax.broadcasted_iota(jnp.int32, (2 * 128, 128), 0)
    kk = lax.broadcasted_iota(jnp.int32, (2 * 128, 128), 1)
    sel = (jj // 2 == kk).astype(jnp.float32)
    idx_f = lax.dot_general(
        w, sel,
        dimension_numbers=(((1,), (0,)), ((), ())),
        preferred_element_type=jnp.float32,
    )
    idx_ref[...] = idx_f.astype(jnp.int32)


def _indices(xr):
    bs = 512
    return pl.pallas_call(
        _idx_body,
        grid=(NROW // bs,),
        in_specs=[pl.BlockSpec((bs, 2 * 128), lambda i: (i, 0))],
        out_specs=pl.BlockSpec((bs, 128), lambda i: (i, 0)),
        out_shape=jax.ShapeDtypeStruct((NROW, 128), jnp.int32),
    )(xr)


def _tc_gather_body(idx_ref, m_ref, out_ref):
    m = m_ref[...]
    for q in range(KC):
        col = idx_ref[0, :, q:q + 1]
        oh = (col == lax.broadcasted_iota(jnp.int32, (128, VOCAB), 1))
        out_ref[q * 128:(q + 1) * 128, :] = lax.dot_general(
            oh.astype(jnp.float32), m,
            dimension_numbers=(((1,), (0,)), ((), ())),
            preferred_element_type=jnp.float32,
        )


def _tc_gather(idxT3, M):
    return pl.pallas_call(
        _tc_gather_body,
        grid=(RT // 128 // KC,),
        in_specs=[
            pl.BlockSpec((1, 128, KC), lambda j: (j, 0, 0)),
            pl.BlockSpec((VOCAB, DIM_OUT), lambda j: (0, 0)),
        ],
        out_specs=pl.BlockSpec((KC * 128, DIM_OUT), lambda j: (j, 0)),
        out_shape=jax.ShapeDtypeStruct((RT, DIM_OUT), jnp.float32),
    )(idxT3, M)


NBUF = 3
CHW = CH * DIM_OUT


@functools.cache
def _sc_gather_kernel():
    @functools.partial(
        pl.kernel,
        mesh=plsc.VectorSubcoreMesh(
            core_axis_name="c", subcore_axis_name="s", num_cores=2
        ),
        out_type=jax.ShapeDtypeStruct((BSC * DIM_OUT,), jnp.float32),
        compiler_params=pltpu.CompilerParams(needs_layout_passes=False),
        scratch_types=[
            pltpu.VMEM((VOCAB * DIM_OUT,), jnp.float32),
            pltpu.VMEM((NCH, CH), jnp.int32),
            *[pltpu.VMEM((CHW,), jnp.float32) for _ in range(NBUF)],
            pltpu.SemaphoreType.DMA,
            *[pltpu.SemaphoreType.DMA for _ in range(NBUF)],
        ],
    )
    def _sc_gather(m_hbm, idx_hbm, out_hbm, m_v, idx_v, *rest):
        bufs = rest[:NBUF]
        lsem = rest[NBUF]
        ssems = rest[NBUF + 1:]
        wid = lax.axis_index("s") * 2 + lax.axis_index("c")
        base = wid * BPW * DIM_OUT
        pltpu.sync_copy(idx_hbm.at[pl.ds(wid * NCH, NCH)], idx_v)
        pltpu.async_copy(m_hbm, m_v, lsem).wait()

        iota = lax.iota(jnp.int32, 16)
        diag = [(iota + k) & 15 for k in range(16)]

        def body(r, carry):
            for b in range(NBUF):
                jj = r * NBUF + b

                @pl.when(r > 0)
                def _():
                    pltpu.make_async_copy(
                        bufs[b],
                        out_hbm.at[pl.ds(base + (jj - NBUF) * CHW, CHW)],
                        ssems[b]).wait()

                buf = bufs[b]

                @plsc.parallel_loop(0, CH // 16, unroll=1)
                def _(g):
                    rows = idx_v[jj, pl.ds(pl.multiple_of(g * 16, 16), 16)]
                    raddr = rows * DIM_OUT
                    waddr = (g * 16 + iota) * DIM_OUT
                    for t in range(DIM_OUT // 16):
                        for k in range(16):
                            c = diag[k] + t * 16
                            v = plsc.load_gather(m_v, [raddr + c])
                            plsc.store_scatter(buf, [waddr + c], v)

                pltpu.async_copy(
                    bufs[b], out_hbm.at[pl.ds(base + jj * CHW, CHW)],
                    ssems[b])
            return carry

        lax.fori_loop(0, NCH // NBUF, body, 0)

        for b in range(NBUF):
            pltpu.make_async_copy(
                bufs[b],
                out_hbm.at[pl.ds(base + (NCH - NBUF + b) * CHW, CHW)],
                ssems[b]).wait()

    return _sc_gather


def kernel(x, table, W):
    M = _fused_table(table, W)
    xr = x.reshape(NROW, 2 * 128)
    idx2 = _indices(xr)
    idx_sc = idx2[RT // 128:]
    sc_out = _sc_gather_kernel()(M.reshape(-1), idx_sc)
    idxT3 = jnp.transpose(
        idx2[:RT // 128].reshape(RT // 128 // KC, KC, 128), (0, 2, 1))
    tc_out = _tc_gather(idxT3, M)
    out = jnp.concatenate([tc_out.reshape(-1), sc_out])
    return out.reshape(BATCH, SEQ, DIM_OUT)
